# Initial kernel scaffold; baseline (speedup 1.0000x reference)
import jax
import jax.numpy as jnp
from jax.experimental import pallas as pl


def kernel(x1, pos1, batch1, x2, pos2, batch2, W1, b1, W2, b2):
    raise NotImplementedError("write your pallas kernel here")


# speedup vs baseline: 16.9231x; 16.9231x over previous
import functools

import jax
import jax.numpy as jnp
from jax import lax
from jax.experimental import pallas as pl
from jax.experimental.pallas import tpu as pltpu
from jax.experimental.pallas import tpu_sc as plsc

_R2 = 0.2 * 0.2
_CAP = 64
_N1, _N2, _D = 2048, 8192, 128
_HID, _OUT = 128, 128
_NB = 8

_L = 16
_NW = 32
_DPT = _N1 // _NW
_QR = _CAP + _L


def _dense_body(x2_ref, p2_ref, p1_ref, w1x_ref, w1p_ref, b1_ref, w2_ref,
                b_ref, q_ref, s_ref):
    w1p = w1p_ref[...]
    b = jnp.dot(x2_ref[...], w1x_ref[...], preferred_element_type=jnp.float32)
    b = b + jnp.dot(p2_ref[...], w1p, preferred_element_type=jnp.float32)
    b = b + b1_ref[...]
    b_ref[...] = b
    q = jnp.dot(p1_ref[...], w1p, preferred_element_type=jnp.float32)
    q_ref[...] = q
    h = jnp.maximum(b[:_N1, :] - q, 0.0)
    s_ref[...] = jnp.dot(h, w2_ref[...], preferred_element_type=jnp.float32)


def _dense_stage(x2, pos2p, pos1p, w1x, w1p, b1, w2):
    return pl.pallas_call(
        _dense_body,
        out_shape=[
            jax.ShapeDtypeStruct((_N2, _HID), jnp.float32),
            jax.ShapeDtypeStruct((_N1, _HID), jnp.float32),
            jax.ShapeDtypeStruct((_N1, _OUT), jnp.float32),
        ],
    )(x2, pos2p, pos1p, w1x, w1p, b1, w2)


def _sc_body(p2x_hbm, p2y_hbm, p2z_hbm, qx_hbm, qy_hbm, qz_hbm,
             qlo_hbm, qhi_hbm, b_hbm,
             g_hbm,
             p2x_v, p2y_v, p2z_v, qx_v, qy_v, qz_v, qlo_v, qhi_v,
             idx_v, rows_v, sem):
    wid = lax.axis_index("s") * 2 + lax.axis_index("c")
    base = wid * _DPT
    ebase = wid * _DPT * _CAP

    pltpu.sync_copy(p2x_hbm, p2x_v)
    pltpu.sync_copy(p2y_hbm, p2y_v)
    pltpu.sync_copy(p2z_hbm, p2z_v)
    pltpu.sync_copy(qx_hbm.at[pl.ds(base, _DPT)], qx_v)
    pltpu.sync_copy(qy_hbm.at[pl.ds(base, _DPT)], qy_v)
    pltpu.sync_copy(qz_hbm.at[pl.ds(base, _DPT)], qz_v)
    pltpu.sync_copy(qlo_hbm.at[pl.ds(base, _DPT)], qlo_v)
    pltpu.sync_copy(qhi_hbm.at[pl.ds(base, _DPT)], qhi_v)

    iota = lax.iota(jnp.int32, _L)

    def blk_body(blk, carry):
        qb = blk * _L
        qx16 = qx_v[pl.ds(qb, _L)]
        qy16 = qy_v[pl.ds(qb, _L)]
        qz16 = qz_v[pl.ds(qb, _L)]
        qlo16 = qlo_v[pl.ds(qb, _L)]
        qhi16 = qhi_v[pl.ds(qb, _L)]
        for lane in range(_L):
            d = qb + lane
            qxs = qx16[lane]
            qys = qy16[lane]
            qzs = qz16[lane]
            lo = qlo16[lane]
            hi = qhi16[lane]
            dbase = d * _QR
            selfrow = base + d
            fill = iota * 0 + selfrow
            for t in range(_CAP // _L):
                idx_v[pl.ds(dbase + t * _L, _L)] = fill
            c0 = lo // _L
            nch = (hi - c0 * _L + _L - 1) // _L

            def cbody(ci, cnt):
                jb = (c0 + ci) * _L
                jvec = jb + iota
                dx = p2x_v[pl.ds(jb, _L)] - qxs
                dy = p2y_v[pl.ds(jb, _L)] - qys
                dz = p2z_v[pl.ds(jb, _L)] - qzs
                d2 = dx * dx + dy * dy + dz * dz
                m = (d2 <= _R2) & (jvec >= lo) & (jvec < hi)
                s = jnp.where(m, 1, 0)
                for k in (1, 2, 4, 8):
                    sh = s.at[jnp.maximum(iota - k, 0)].get(
                        mode="promise_in_bounds")
                    s = s + jnp.where(iota >= k, sh, 0)
                count = s[_L - 1]

                @pl.when(count > 0)
                def _():
                    tgt = iota + 1
                    pos = iota * 0
                    for st in (8, 4, 2, 1):
                        nxt = pos + st
                        val = s.at[nxt - 1].get(mode="promise_in_bounds")
                        pos = jnp.where(val < tgt, nxt, pos)
                    comp = jvec.at[jnp.minimum(pos, _L - 1)].get(
                        mode="promise_in_bounds")
                    res = jnp.where(iota < count, comp, selfrow)
                    idx_v[pl.ds(dbase + jnp.minimum(cnt, _CAP), _L)] = res

                return cnt + count

            lax.fori_loop(0, nch, cbody, jnp.int32(0))
        return carry

    lax.fori_loop(0, _DPT // _L, blk_body, 0)

    def gather_body(d, carry):
        pltpu.async_copy(b_hbm.at[idx_v.at[pl.ds(d * _QR, _CAP)]],
                         rows_v, sem).wait()
        pltpu.sync_copy(rows_v, g_hbm.at[pl.ds(ebase + d * _CAP, _CAP)])
        return carry

    lax.fori_loop(0, _DPT, gather_body, 0)


def _sc_stage(p2x, p2y, p2z, qx, qy, qz, qlo, qhi, b):
    mesh = plsc.VectorSubcoreMesh(core_axis_name="c", subcore_axis_name="s",
                                  num_cores=2, num_subcores=16)
    f = functools.partial(
        pl.kernel,
        out_type=jax.ShapeDtypeStruct((_N1 * _CAP, _HID), jnp.float32),
        mesh=mesh,
        scratch_types=[
            pltpu.VMEM((_N2,), jnp.float32),
            pltpu.VMEM((_N2,), jnp.float32),
            pltpu.VMEM((_N2,), jnp.float32),
            pltpu.VMEM((_DPT,), jnp.float32),
            pltpu.VMEM((_DPT,), jnp.float32),
            pltpu.VMEM((_DPT,), jnp.float32),
            pltpu.VMEM((_DPT,), jnp.int32),
            pltpu.VMEM((_DPT,), jnp.int32),
            pltpu.VMEM((_DPT * _QR,), jnp.int32),
            pltpu.VMEM((_CAP, _HID), jnp.float32),
            pltpu.SemaphoreType.DMA,
        ],
    )(_sc_body)
    return f(p2x, p2y, p2z, qx, qy, qz, qlo, qhi, b)


_TILE = 128


def _agg_body(g_ref, q_ref, s_ref, w2_ref, b2_ref, out_ref):
    g = g_ref[...].reshape(_TILE, _CAP, _HID)
    q = q_ref[...]
    h = jnp.maximum(g - q[:, None, :], 0.0).reshape(_TILE * _CAP, _HID)
    m = jnp.dot(h, w2_ref[...], preferred_element_type=jnp.float32)
    m = m.reshape(_TILE, _CAP, _OUT)
    r = jnp.max(m, axis=1)
    out_ref[...] = jnp.maximum(r, s_ref[...]) + b2_ref[...]


def _agg_stage(g, q, s, w2, b2):
    nsteps = _N1 // _TILE
    return pl.pallas_call(
        _agg_body,
        grid=(nsteps,),
        in_specs=[
            pl.BlockSpec((_TILE * _CAP, _HID), lambda i: (i, 0)),
            pl.BlockSpec((_TILE, _HID), lambda i: (i, 0)),
            pl.BlockSpec((_TILE, _OUT), lambda i: (i, 0)),
            pl.BlockSpec((_HID, _OUT), lambda i: (0, 0)),
            pl.BlockSpec((1, _OUT), lambda i: (0, 0)),
        ],
        out_specs=pl.BlockSpec((_TILE, _OUT), lambda i: (i, 0)),
        out_shape=jax.ShapeDtypeStruct((_N1, _OUT), jnp.float32),
    )(g, q, s, w2, b2)


def kernel(x1, pos1, batch1, x2, pos2, batch2, W1, b1, W2, b2):
    del x1
    pos1 = pos1.astype(jnp.float32)
    pos2 = pos2.astype(jnp.float32)
    b1i = batch1.astype(jnp.int32)
    b2i = batch2.astype(jnp.int32)

    w1x = W1[:_D, :]
    w1p = jnp.zeros((8, _HID), jnp.float32).at[:3, :].set(W1[_D:, :])
    pos1p = jnp.zeros((_N1, 8), jnp.float32).at[:, :3].set(pos1)
    pos2p = jnp.zeros((_N2, 8), jnp.float32).at[:, :3].set(pos2)

    B, Q, S = _dense_stage(x2.astype(jnp.float32), pos2p, pos1p,
                           w1x, w1p, b1.reshape(1, _HID), W2)

    nb = jnp.arange(_NB, dtype=b2i.dtype)
    starts = jnp.searchsorted(b2i, nb, side="left").astype(jnp.int32)
    ends = jnp.searchsorted(b2i, nb, side="right").astype(jnp.int32)
    qlo = starts[b1i]
    qhi = ends[b1i]

    G = _sc_stage(pos2[:, 0], pos2[:, 1], pos2[:, 2],
                  pos1[:, 0], pos1[:, 1], pos1[:, 2],
                  qlo, qhi, B)

    out = _agg_stage(G, Q, S, W2, b2.reshape(1, _OUT))
    return (out, pos1, batch1)


# speedup vs baseline: 21.9387x; 1.2964x over previous
import functools

import jax
import jax.numpy as jnp
from jax import lax
from jax.experimental import pallas as pl
from jax.experimental.pallas import tpu as pltpu
from jax.experimental.pallas import tpu_sc as plsc

_R2 = 0.2 * 0.2
_CAP = 64
_N1, _N2, _D = 2048, 8192, 128
_HID, _OUT = 128, 128
_NB = 8

_L = 16
_NW = 32
_DPT = _N1 // _NW
_QR = _CAP + _L


def _dense_body(x2_ref, p2_ref, p1_ref, w1x_ref, w1p_ref, b1_ref, w2_ref,
                b_ref, q_ref, s_ref):
    w1p = w1p_ref[...]
    b = jnp.dot(x2_ref[...], w1x_ref[...], preferred_element_type=jnp.float32)
    b = b + jnp.dot(p2_ref[...], w1p, preferred_element_type=jnp.float32)
    b = b + b1_ref[...]
    b_ref[...] = b
    q = jnp.dot(p1_ref[...], w1p, preferred_element_type=jnp.float32)
    q_ref[...] = q
    h = jnp.maximum(b[:_N1, :] - q, 0.0)
    s_ref[...] = jnp.dot(h, w2_ref[...], preferred_element_type=jnp.float32)


def _dense_stage(x2, pos2p, pos1p, w1x, w1p, b1, w2):
    return pl.pallas_call(
        _dense_body,
        out_shape=[
            jax.ShapeDtypeStruct((_N2, _HID), jnp.float32),
            jax.ShapeDtypeStruct((_N1, _HID), jnp.float32),
            jax.ShapeDtypeStruct((_N1, _OUT), jnp.float32),
        ],
    )(x2, pos2p, pos1p, w1x, w1p, b1, w2)


def _sc_body(p2x_hbm, p2y_hbm, p2z_hbm, qx_hbm, qy_hbm, qz_hbm,
             qlo_hbm, qhi_hbm, b_hbm,
             g_hbm,
             p2x_v, p2y_v, p2z_v, qx_v, qy_v, qz_v, qlo_v, qhi_v,
             idx_v, rows0_v, rows1_v, rows2_v, rows3_v,
             gsem0, gsem1, gsem2, gsem3, wsem0, wsem1, wsem2, wsem3):
    wid = lax.axis_index("s") * 2 + lax.axis_index("c")
    base = wid * _DPT
    ebase = wid * _DPT * _CAP

    pltpu.sync_copy(p2x_hbm, p2x_v)
    pltpu.sync_copy(p2y_hbm, p2y_v)
    pltpu.sync_copy(p2z_hbm, p2z_v)
    pltpu.sync_copy(qx_hbm.at[pl.ds(base, _DPT)], qx_v)
    pltpu.sync_copy(qy_hbm.at[pl.ds(base, _DPT)], qy_v)
    pltpu.sync_copy(qz_hbm.at[pl.ds(base, _DPT)], qz_v)
    pltpu.sync_copy(qlo_hbm.at[pl.ds(base, _DPT)], qlo_v)
    pltpu.sync_copy(qhi_hbm.at[pl.ds(base, _DPT)], qhi_v)

    iota = lax.iota(jnp.int32, _L)

    def blk_body(blk, carry):
        qb = blk * _L
        qx16 = qx_v[pl.ds(qb, _L)]
        qy16 = qy_v[pl.ds(qb, _L)]
        qz16 = qz_v[pl.ds(qb, _L)]
        qlo16 = qlo_v[pl.ds(qb, _L)]
        qhi16 = qhi_v[pl.ds(qb, _L)]
        for lane in range(_L):
            d = qb + lane
            qxs = qx16[lane]
            qys = qy16[lane]
            qzs = qz16[lane]
            lo = qlo16[lane]
            hi = qhi16[lane]
            dbase = d * _QR
            selfrow = base + d
            fill = iota * 0 + selfrow
            for t in range(_CAP // _L):
                idx_v[pl.ds(dbase + t * _L, _L)] = fill
            c0 = lo // _L
            nch = (hi - c0 * _L + _L - 1) // _L

            def cbody(ci, cnt):
                jb = (c0 + ci) * _L
                jvec = jb + iota
                dx = p2x_v[pl.ds(jb, _L)] - qxs
                dy = p2y_v[pl.ds(jb, _L)] - qys
                dz = p2z_v[pl.ds(jb, _L)] - qzs
                d2 = dx * dx + dy * dy + dz * dz
                m = (d2 <= _R2) & (jvec >= lo) & (jvec < hi)
                s = jnp.where(m, 1, 0)
                for k in (1, 2, 4, 8):
                    sh = s.at[jnp.maximum(iota - k, 0)].get(
                        mode="promise_in_bounds")
                    s = s + jnp.where(iota >= k, sh, 0)
                count = s[_L - 1]

                @pl.when(count > 0)
                def _():
                    tgt = iota + 1
                    pos = iota * 0
                    for st in (8, 4, 2, 1):
                        nxt = pos + st
                        val = s.at[nxt - 1].get(mode="promise_in_bounds")
                        pos = jnp.where(val < tgt, nxt, pos)
                    comp = jvec.at[jnp.minimum(pos, _L - 1)].get(
                        mode="promise_in_bounds")
                    res = jnp.where(iota < count, comp, selfrow)
                    idx_v[pl.ds(dbase + jnp.minimum(cnt, _CAP), _L)] = res

                return cnt + count

            lax.fori_loop(0, nch, cbody, jnp.int32(0))
        return carry

    lax.fori_loop(0, _DPT // _L, blk_body, 0)

    bufs = (rows0_v, rows1_v, rows2_v, rows3_v)
    gsems = (gsem0, gsem1, gsem2, gsem3)
    wsems = (wsem0, wsem1, wsem2, wsem3)

    def g_desc(d, j):
        return pltpu.make_async_copy(
            b_hbm.at[idx_v.at[pl.ds(d * _QR, _CAP)]], bufs[j], gsems[j])

    def w_desc(d, j):
        return pltpu.make_async_copy(
            bufs[j], g_hbm.at[pl.ds(ebase + d * _CAP, _CAP)], wsems[j])

    for j in range(4):
        g_desc(j, j).start()

    def pipe_body(k, carry):
        for j in range(4):
            d = 4 * k + j
            g_desc(d, j).wait()
            w_desc(d, j).start()
        for j in range(4):
            d = 4 * k + j
            w_desc(d, j).wait()
            g_desc(d + 4, j).start()
        return carry

    lax.fori_loop(0, _DPT // 4 - 1, pipe_body, 0)

    for j in range(4):
        d = _DPT - 4 + j
        g_desc(d, j).wait()
        w_desc(d, j).start()
    for j in range(4):
        w_desc(_DPT - 4 + j, j).wait()


def _sc_stage(p2x, p2y, p2z, qx, qy, qz, qlo, qhi, b):
    mesh = plsc.VectorSubcoreMesh(core_axis_name="c", subcore_axis_name="s",
                                  num_cores=2, num_subcores=16)
    f = functools.partial(
        pl.kernel,
        out_type=jax.ShapeDtypeStruct((_N1 * _CAP, _HID), jnp.float32),
        mesh=mesh,
        scratch_types=[
            pltpu.VMEM((_N2,), jnp.float32),
            pltpu.VMEM((_N2,), jnp.float32),
            pltpu.VMEM((_N2,), jnp.float32),
            pltpu.VMEM((_DPT,), jnp.float32),
            pltpu.VMEM((_DPT,), jnp.float32),
            pltpu.VMEM((_DPT,), jnp.float32),
            pltpu.VMEM((_DPT,), jnp.int32),
            pltpu.VMEM((_DPT,), jnp.int32),
            pltpu.VMEM((_DPT * _QR,), jnp.int32),
            pltpu.VMEM((_CAP, _HID), jnp.float32),
            pltpu.VMEM((_CAP, _HID), jnp.float32),
            pltpu.VMEM((_CAP, _HID), jnp.float32),
            pltpu.VMEM((_CAP, _HID), jnp.float32),
            pltpu.SemaphoreType.DMA,
            pltpu.SemaphoreType.DMA,
            pltpu.SemaphoreType.DMA,
            pltpu.SemaphoreType.DMA,
            pltpu.SemaphoreType.DMA,
            pltpu.SemaphoreType.DMA,
            pltpu.SemaphoreType.DMA,
            pltpu.SemaphoreType.DMA,
        ],
    )(_sc_body)
    return f(p2x, p2y, p2z, qx, qy, qz, qlo, qhi, b)


_TILE = 128


def _agg_body(g_ref, q_ref, s_ref, w2_ref, b2_ref, out_ref):
    g = g_ref[...].reshape(_TILE, _CAP, _HID)
    q = q_ref[...]
    h = jnp.maximum(g - q[:, None, :], 0.0).reshape(_TILE * _CAP, _HID)
    m = jnp.dot(h, w2_ref[...], preferred_element_type=jnp.float32)
    m = m.reshape(_TILE, _CAP, _OUT)
    r = jnp.max(m, axis=1)
    out_ref[...] = jnp.maximum(r, s_ref[...]) + b2_ref[...]


def _agg_stage(g, q, s, w2, b2):
    nsteps = _N1 // _TILE
    return pl.pallas_call(
        _agg_body,
        grid=(nsteps,),
        in_specs=[
            pl.BlockSpec((_TILE * _CAP, _HID), lambda i: (i, 0)),
            pl.BlockSpec((_TILE, _HID), lambda i: (i, 0)),
            pl.BlockSpec((_TILE, _OUT), lambda i: (i, 0)),
            pl.BlockSpec((_HID, _OUT), lambda i: (0, 0)),
            pl.BlockSpec((1, _OUT), lambda i: (0, 0)),
        ],
        out_specs=pl.BlockSpec((_TILE, _OUT), lambda i: (i, 0)),
        out_shape=jax.ShapeDtypeStruct((_N1, _OUT), jnp.float32),
    )(g, q, s, w2, b2)


def kernel(x1, pos1, batch1, x2, pos2, batch2, W1, b1, W2, b2):
    del x1
    pos1 = pos1.astype(jnp.float32)
    pos2 = pos2.astype(jnp.float32)
    b1i = batch1.astype(jnp.int32)
    b2i = batch2.astype(jnp.int32)

    w1x = W1[:_D, :]
    w1p = jnp.zeros((8, _HID), jnp.float32).at[:3, :].set(W1[_D:, :])
    pos1p = jnp.zeros((_N1, 8), jnp.float32).at[:, :3].set(pos1)
    pos2p = jnp.zeros((_N2, 8), jnp.float32).at[:, :3].set(pos2)

    B, Q, S = _dense_stage(x2.astype(jnp.float32), pos2p, pos1p,
                           w1x, w1p, b1.reshape(1, _HID), W2)

    nb = jnp.arange(_NB, dtype=b2i.dtype)
    starts = jnp.searchsorted(b2i, nb, side="left").astype(jnp.int32)
    ends = jnp.searchsorted(b2i, nb, side="right").astype(jnp.int32)
    qlo = starts[b1i]
    qhi = ends[b1i]

    G = _sc_stage(pos2[:, 0], pos2[:, 1], pos2[:, 2],
                  pos1[:, 0], pos1[:, 1], pos1[:, 2],
                  qlo, qhi, B)

    out = _agg_stage(G, Q, S, W2, b2.reshape(1, _OUT))
    return (out, pos1, batch1)


# speedup vs baseline: 30.9592x; 1.4112x over previous
import functools

import jax
import jax.numpy as jnp
from jax import lax
from jax.experimental import pallas as pl
from jax.experimental.pallas import tpu as pltpu
from jax.experimental.pallas import tpu_sc as plsc

_R2 = 0.2 * 0.2
_CAP = 64
_N1, _N2, _D = 2048, 8192, 128
_HID, _OUT = 128, 128
_NB = 8

_L = 16
_NW = 32
_DPT = _N1 // _NW
_QR = _CAP + _L


def _dense_body(x2_ref, p2_ref, p1_ref, w1x_ref, w1p_ref, b1_ref, w2_ref,
                b_ref, q_ref, s_ref):
    w1p = w1p_ref[...]
    b = jnp.dot(x2_ref[...], w1x_ref[...], preferred_element_type=jnp.float32)
    b = b + jnp.dot(p2_ref[...], w1p, preferred_element_type=jnp.float32)
    b = b + b1_ref[...]
    b_ref[...] = b
    q = jnp.dot(p1_ref[...], w1p, preferred_element_type=jnp.float32)
    q_ref[...] = q
    h = jnp.maximum(b[:_N1, :] - q, 0.0)
    s_ref[...] = jnp.dot(h, w2_ref[...], preferred_element_type=jnp.float32)


def _dense_stage(x2, pos2p, pos1p, w1x, w1p, b1, w2):
    return pl.pallas_call(
        _dense_body,
        out_shape=[
            jax.ShapeDtypeStruct((_N2, _HID), jnp.float32),
            jax.ShapeDtypeStruct((_N1, _HID), jnp.float32),
            jax.ShapeDtypeStruct((_N1, _OUT), jnp.float32),
        ],
    )(x2, pos2p, pos1p, w1x, w1p, b1, w2)


def _sc_body(p2x_hbm, p2y_hbm, p2z_hbm, qx_hbm, qy_hbm, qz_hbm,
             qlo_hbm, qhi_hbm, b_hbm,
             g_hbm,
             p2x_v, p2y_v, p2z_v, qx_v, qy_v, qz_v, qlo_v, qhi_v,
             idx_v, rows0_v, rows1_v, rows2_v, rows3_v,
             gsem0, gsem1, gsem2, gsem3, wsem0, wsem1, wsem2, wsem3):
    wid = lax.axis_index("s") * 2 + lax.axis_index("c")
    base = wid * _DPT
    ebase = wid * _DPT * _CAP

    pltpu.sync_copy(p2x_hbm, p2x_v)
    pltpu.sync_copy(p2y_hbm, p2y_v)
    pltpu.sync_copy(p2z_hbm, p2z_v)
    pltpu.sync_copy(qx_hbm.at[pl.ds(base, _DPT)], qx_v)
    pltpu.sync_copy(qy_hbm.at[pl.ds(base, _DPT)], qy_v)
    pltpu.sync_copy(qz_hbm.at[pl.ds(base, _DPT)], qz_v)
    pltpu.sync_copy(qlo_hbm.at[pl.ds(base, _DPT)], qlo_v)
    pltpu.sync_copy(qhi_hbm.at[pl.ds(base, _DPT)], qhi_v)

    iota = lax.iota(jnp.int32, _L)

    bufs = (rows0_v, rows1_v, rows2_v, rows3_v)
    gsems = (gsem0, gsem1, gsem2, gsem3)
    wsems = (wsem0, wsem1, wsem2, wsem3)

    def g_desc(d, j):
        return pltpu.make_async_copy(
            b_hbm.at[idx_v.at[pl.ds(d * _QR, _CAP)]], bufs[j], gsems[j])

    def w_desc(d, j):
        return pltpu.make_async_copy(
            bufs[j], g_hbm.at[pl.ds(ebase + d * _CAP, _CAP)], wsems[j])

    def search_block(qb, ring):
        qx16 = qx_v[pl.ds(qb, _L)]
        qy16 = qy_v[pl.ds(qb, _L)]
        qz16 = qz_v[pl.ds(qb, _L)]
        qlo16 = qlo_v[pl.ds(qb, _L)]
        qhi16 = qhi_v[pl.ds(qb, _L)]
        for lane in range(_L):
            d = qb + lane
            qxs = qx16[lane]
            qys = qy16[lane]
            qzs = qz16[lane]
            lo = qlo16[lane]
            hi = qhi16[lane]
            dbase = d * _QR
            selfrow = base + d
            fill = iota * 0 + selfrow
            for t in range(_CAP // _L):
                idx_v[pl.ds(dbase + t * _L, _L)] = fill
            c0 = lo // _L
            nch = (hi - c0 * _L + _L - 1) // _L

            def cbody(ci, cnt):
                jb = (c0 + ci) * _L
                jvec = jb + iota
                dx = p2x_v[pl.ds(jb, _L)] - qxs
                dy = p2y_v[pl.ds(jb, _L)] - qys
                dz = p2z_v[pl.ds(jb, _L)] - qzs
                d2 = dx * dx + dy * dy + dz * dz
                m = (d2 <= _R2) & (jvec >= lo) & (jvec < hi)
                s = jnp.where(m, 1, 0)
                for k in (1, 2, 4, 8):
                    sh = s.at[jnp.maximum(iota - k, 0)].get(
                        mode="promise_in_bounds")
                    s = s + jnp.where(iota >= k, sh, 0)
                count = s[_L - 1]

                @pl.when(count > 0)
                def _():
                    tgt = iota + 1
                    pos = iota * 0
                    for st in (8, 4, 2, 1):
                        nxt = pos + st
                        val = s.at[nxt - 1].get(mode="promise_in_bounds")
                        pos = jnp.where(val < tgt, nxt, pos)
                    comp = jvec.at[jnp.minimum(pos, _L - 1)].get(
                        mode="promise_in_bounds")
                    res = jnp.where(iota < count, comp, selfrow)
                    idx_v[pl.ds(dbase + jnp.minimum(cnt, _CAP), _L)] = res

                return cnt + count

            lax.fori_loop(0, nch, cbody, jnp.int32(0))
            ring(lane, d)

    def ring_first(lane, d):
        j = lane & 3
        if lane >= 4:
            w_desc(d - 4, j).wait()
        g_desc(d, j).start()
        if lane >= 3:
            j3 = (lane - 3) & 3
            g_desc(d - 3, j3).wait()
            w_desc(d - 3, j3).start()

    def ring_steady(lane, d):
        j = lane & 3
        w_desc(d - 4, j).wait()
        g_desc(d, j).start()
        j3 = (lane - 3) & 3
        g_desc(d - 3, j3).wait()
        w_desc(d - 3, j3).start()

    search_block(0, ring_first)

    def blk_body(blk, carry):
        search_block(blk * _L, ring_steady)
        return carry

    lax.fori_loop(1, _DPT // _L, blk_body, 0)

    for d in range(_DPT - 3, _DPT):
        g_desc(d, d & 3).wait()
        w_desc(d, d & 3).start()
    for d in range(_DPT - 4, _DPT):
        w_desc(d, d & 3).wait()


def _sc_stage(p2x, p2y, p2z, qx, qy, qz, qlo, qhi, b):
    mesh = plsc.VectorSubcoreMesh(core_axis_name="c", subcore_axis_name="s",
                                  num_cores=2, num_subcores=16)
    f = functools.partial(
        pl.kernel,
        out_type=jax.ShapeDtypeStruct((_N1 * _CAP, _HID), jnp.float32),
        mesh=mesh,
        scratch_types=[
            pltpu.VMEM((_N2,), jnp.float32),
            pltpu.VMEM((_N2,), jnp.float32),
            pltpu.VMEM((_N2,), jnp.float32),
            pltpu.VMEM((_DPT,), jnp.float32),
            pltpu.VMEM((_DPT,), jnp.float32),
            pltpu.VMEM((_DPT,), jnp.float32),
            pltpu.VMEM((_DPT,), jnp.int32),
            pltpu.VMEM((_DPT,), jnp.int32),
            pltpu.VMEM((_DPT * _QR,), jnp.int32),
            pltpu.VMEM((_CAP, _HID), jnp.float32),
            pltpu.VMEM((_CAP, _HID), jnp.float32),
            pltpu.VMEM((_CAP, _HID), jnp.float32),
            pltpu.VMEM((_CAP, _HID), jnp.float32),
            pltpu.SemaphoreType.DMA,
            pltpu.SemaphoreType.DMA,
            pltpu.SemaphoreType.DMA,
            pltpu.SemaphoreType.DMA,
            pltpu.SemaphoreType.DMA,
            pltpu.SemaphoreType.DMA,
            pltpu.SemaphoreType.DMA,
            pltpu.SemaphoreType.DMA,
        ],
    )(_sc_body)
    return f(p2x, p2y, p2z, qx, qy, qz, qlo, qhi, b)


_TILE = 128


def _agg_body(g_ref, q_ref, s_ref, w2_ref, b2_ref, out_ref):
    g = g_ref[...].reshape(_TILE, _CAP, _HID)
    q = q_ref[...]
    h = jnp.maximum(g - q[:, None, :], 0.0).reshape(_TILE * _CAP, _HID)
    m = jnp.dot(h, w2_ref[...], preferred_element_type=jnp.float32)
    m = m.reshape(_TILE, _CAP, _OUT)
    r = jnp.max(m, axis=1)
    out_ref[...] = jnp.maximum(r, s_ref[...]) + b2_ref[...]


def _agg_stage(g, q, s, w2, b2):
    nsteps = _N1 // _TILE
    return pl.pallas_call(
        _agg_body,
        grid=(nsteps,),
        in_specs=[
            pl.BlockSpec((_TILE * _CAP, _HID), lambda i: (i, 0)),
            pl.BlockSpec((_TILE, _HID), lambda i: (i, 0)),
            pl.BlockSpec((_TILE, _OUT), lambda i: (i, 0)),
            pl.BlockSpec((_HID, _OUT), lambda i: (0, 0)),
            pl.BlockSpec((1, _OUT), lambda i: (0, 0)),
        ],
        out_specs=pl.BlockSpec((_TILE, _OUT), lambda i: (i, 0)),
        out_shape=jax.ShapeDtypeStruct((_N1, _OUT), jnp.float32),
    )(g, q, s, w2, b2)


def kernel(x1, pos1, batch1, x2, pos2, batch2, W1, b1, W2, b2):
    del x1
    pos1 = pos1.astype(jnp.float32)
    pos2 = pos2.astype(jnp.float32)
    b1i = batch1.astype(jnp.int32)
    b2i = batch2.astype(jnp.int32)

    w1x = W1[:_D, :]
    w1p = jnp.zeros((8, _HID), jnp.float32).at[:3, :].set(W1[_D:, :])
    pos1p = jnp.zeros((_N1, 8), jnp.float32).at[:, :3].set(pos1)
    pos2p = jnp.zeros((_N2, 8), jnp.float32).at[:, :3].set(pos2)

    B, Q, S = _dense_stage(x2.astype(jnp.float32), pos2p, pos1p,
                           w1x, w1p, b1.reshape(1, _HID), W2)

    nb = jnp.arange(_NB, dtype=b2i.dtype)
    starts = jnp.searchsorted(b2i, nb, side="left").astype(jnp.int32)
    ends = jnp.searchsorted(b2i, nb, side="right").astype(jnp.int32)
    qlo = starts[b1i]
    qhi = ends[b1i]

    G = _sc_stage(pos2[:, 0], pos2[:, 1], pos2[:, 2],
                  pos1[:, 0], pos1[:, 1], pos1[:, 2],
                  qlo, qhi, B)

    out = _agg_stage(G, Q, S, W2, b2.reshape(1, _OUT))
    return (out, pos1, batch1)
